# baseline (speedup 1.0000x reference)
import functools

import jax
import jax.numpy as jnp
from jax import lax
from jax.experimental import pallas as pl
from jax.experimental.pallas import tpu as pltpu
from jax.experimental.pallas import tpu_sc as plsc

_D = 32
_H1 = 64
_H2 = 32


def _detile_table(table):
    V = table.shape[0]
    VF = (V // 512) * 512
    tableT = table.T
    tail = table[VF:].reshape(((V - VF) * _D) // 128, 128)
    n_tail = tail.shape[0]
    NG = VF // 512
    R4 = (V * _D) // 128
    info = plsc.get_sparse_core_info()
    NC, NS = info.num_cores, info.num_subcores
    NW = NC * NS
    per_w = 2 * (-(-NG // (2 * NW)))
    mesh = plsc.VectorSubcoreMesh(core_axis_name="c", subcore_axis_name="s")

    @functools.partial(
        pl.kernel,
        out_type=jax.ShapeDtypeStruct((R4, 128), jnp.float32),
        mesh=mesh,
        scratch_types=[
            pltpu.VMEM((128, 128), jnp.float32),
            pltpu.VMEM((128, 128), jnp.float32),
            pltpu.VMEM((128, 128), jnp.float32),
            pltpu.VMEM((128, 128), jnp.float32),
            pltpu.SemaphoreType.DMA,
            pltpu.SemaphoreType.DMA,
            pltpu.SemaphoreType.DMA,
            pltpu.SemaphoreType.DMA,
        ],
        compiler_params=pltpu.CompilerParams(
            use_tc_tiling_on_sc=True, needs_layout_passes=False
        ),
    )
    def transpose_kernel(
        tt_hbm, tail_hbm, t4_hbm, inA, inB, outA, outB, isemA, isemB,
        osemA, osemB
    ):
        wid = lax.axis_index("s") * NC + lax.axis_index("c")
        iota = lax.iota(jnp.int32, 16)
        tvecs = [(iota + d) & 15 for d in range(16)]
        wbase = [t * 32 + iota for t in tvecs]

        def fire_in(gid, inb, isem):
            base = gid * 512
            for j in range(4):
                pltpu.async_copy(
                    tt_hbm.at[:, pl.ds(base + j * 128, 128)],
                    inb.at[pl.ds(j * 32, 32)],
                    isem,
                )

        def drain_in(gid, inb, isem):
            base = gid * 512
            for j in range(4):
                pltpu.make_async_copy(
                    tt_hbm.at[:, pl.ds(base + j * 128, 128)],
                    inb.at[pl.ds(j * 32, 32)],
                    isem,
                ).wait()

        def extract(inb, outb):
            def emit(m, carry):
                j = m >> 4
                h = (m >> 3) & 1
                e0 = (m & 7) * 16
                rows = j * 32 + h * 16 + iota
                wb = j * 4096 + e0 * 32 + h * 16
                for d in range(16):
                    v = plsc.load_gather(inb, [rows, e0 + tvecs[d]])
                    w = wb + wbase[d]
                    plsc.store_scatter(outb, [w >> 7, w & 127], v)
                return carry

            lax.fori_loop(0, 64, emit, 0)

        def pair_body(i, carry):
            gidA = wid * per_w + 2 * i
            gidB = gidA + 1
            okA, okB = gidA < NG, gidB < NG

            @pl.when(okA)
            def _():
                fire_in(gidA, inA, isemA)

            @pl.when(okB)
            def _():
                fire_in(gidB, inB, isemB)

            @pl.when(okA)
            def _():
                drain_in(gidA, inA, isemA)
                extract(inA, outA)
                pltpu.async_copy(
                    outA, t4_hbm.at[pl.ds(gidA * 128, 128)], osemA
                )

            @pl.when(okB)
            def _():
                drain_in(gidB, inB, isemB)
                extract(inB, outB)
                pltpu.async_copy(
                    outB, t4_hbm.at[pl.ds(gidB * 128, 128)], osemB
                )

            @pl.when(okA)
            def _():
                pltpu.make_async_copy(
                    outA, t4_hbm.at[pl.ds(gidA * 128, 128)], osemA
                ).wait()

            @pl.when(okB)
            def _():
                pltpu.make_async_copy(
                    outB, t4_hbm.at[pl.ds(gidB * 128, 128)], osemB
                ).wait()

            return carry

        lax.fori_loop(0, per_w // 2, pair_body, 0)

        @pl.when(wid == NW - 1)
        def _():
            pltpu.sync_copy(tail_hbm, outA.at[pl.ds(0, n_tail)])
            pltpu.sync_copy(
                outA.at[pl.ds(0, n_tail)], t4_hbm.at[pl.ds(NG * 128, n_tail)]
            )

    return transpose_kernel(tableT, tail)


def _gather_rows(table, idx_flat):
    BF = idx_flat.shape[0]
    info = plsc.get_sparse_core_info()
    NC, NS = info.num_cores, info.num_subcores
    NW = NC * NS
    per_w = BF // NW
    C = 3328
    assert per_w % C == 0
    n_chunks = per_w // C
    mesh = plsc.VectorSubcoreMesh(core_axis_name="c", subcore_axis_name="s")

    @functools.partial(
        pl.kernel,
        out_type=jax.ShapeDtypeStruct((BF, _D), jnp.float32),
        mesh=mesh,
        scratch_types=[
            pltpu.VMEM((C,), jnp.int32),
            pltpu.VMEM((C, _D), jnp.float32),
            pltpu.SemaphoreType.DMA,
        ],
        compiler_params=pltpu.CompilerParams(use_tc_tiling_on_sc=False),
    )
    def gather_kernel(idx_hbm, table_hbm, out_hbm, idx_v, rows_v, sem):
        wid = lax.axis_index("s") * NC + lax.axis_index("c")
        for i in range(n_chunks):
            base = wid * per_w + i * C
            pltpu.sync_copy(idx_hbm.at[pl.ds(base, C)], idx_v)
            pltpu.async_copy(table_hbm.at[idx_v], rows_v, sem).wait()
            pltpu.sync_copy(rows_v, out_hbm.at[pl.ds(base, C)])

    return gather_kernel(idx_flat, table)


def _mlp_transposed(g4, W1d, b1d, W2d, b2t, F, B):
    Q = B // 4

    def body(g_ref, w1_ref, b1_ref, w2_ref, b2_ref, o_ref):
        g = g_ref[0]
        h = jnp.dot(g, w1_ref[...], preferred_element_type=jnp.float32)
        h = jnp.maximum(h + b1_ref[...], 0.0)
        ot = jax.lax.dot_general(
            w2_ref[...], h, (((0,), (1,)), ((), ())),
            preferred_element_type=jnp.float32,
        )
        ot = ot + b2_ref[...]
        for r in range(4):
            o_ref[0, :, r * Q : (r + 1) * Q] = ot[r * _H2 : (r + 1) * _H2, :]

    return pl.pallas_call(
        body,
        grid=(F,),
        in_specs=[
            pl.BlockSpec((1, Q, 4 * _D), lambda f: (f, 0, 0)),
            pl.BlockSpec((4 * _D, 4 * _H1), lambda f: (0, 0)),
            pl.BlockSpec((1, 4 * _H1), lambda f: (0, 0)),
            pl.BlockSpec((4 * _H1, 4 * _H2), lambda f: (0, 0)),
            pl.BlockSpec((4 * _H2, 1), lambda f: (0, 0)),
        ],
        out_specs=pl.BlockSpec((1, _H2, B), lambda f: (f, 0, 0)),
        out_shape=jax.ShapeDtypeStruct((F, _H2, B), jnp.float32),
    )(g4, W1d, b1d, W2d, b2t)


def _block_diag4(W):
    a, b = W.shape
    out = jnp.zeros((4 * a, 4 * b), dtype=W.dtype)
    for r in range(4):
        out = out.at[r * a : (r + 1) * a, r * b : (r + 1) * b].set(W)
    return out


def kernel(x, table, W1, b1, W2, b2):
    B, F = x.shape
    Q = B // 4
    idx_flat = (
        x.T.reshape(F, 4, Q).transpose(0, 2, 1).reshape(-1).astype(jnp.int32)
    )
    table_rm = _detile_table(table).reshape(table.shape[0], _D)
    g = _gather_rows(table_rm, idx_flat)
    g4 = g.reshape(F, Q, 4 * _D)
    W1d = _block_diag4(W1)
    W2d = _block_diag4(W2)
    b1d = jnp.tile(b1, 4).reshape(1, 4 * _H1)
    b2t = jnp.tile(b2, 4).reshape(4 * _H2, 1)
    o3 = _mlp_transposed(g4, W1d, b1d, W2d, b2t, F, B)
    return o3.transpose(2, 0, 1)


# speedup vs baseline: 1.5366x; 1.5366x over previous
import functools

import jax
import jax.numpy as jnp
from jax import lax
from jax.experimental import pallas as pl
from jax.experimental.pallas import tpu as pltpu
from jax.experimental.pallas import tpu_sc as plsc

_D = 32
_H1 = 64
_H2 = 32


def _detile_table(table):
    V = table.shape[0]
    VF = (V // 512) * 512
    tableT = table.T
    tail = table[VF:].reshape(((V - VF) * _D) // 128, 128)
    n_tail = tail.shape[0]
    NG = VF // 512
    R4 = (V * _D) // 128
    info = plsc.get_sparse_core_info()
    NC, NS = info.num_cores, info.num_subcores
    NW = NC * NS
    per_w = 2 * (-(-NG // (2 * NW)))
    mesh = plsc.VectorSubcoreMesh(core_axis_name="c", subcore_axis_name="s")

    @functools.partial(
        pl.kernel,
        out_type=jax.ShapeDtypeStruct((R4, 128), jnp.float32),
        mesh=mesh,
        scratch_types=[
            pltpu.VMEM((128, 128), jnp.float32),
            pltpu.VMEM((128, 128), jnp.float32),
            pltpu.VMEM((128, 128), jnp.float32),
            pltpu.VMEM((128, 128), jnp.float32),
            pltpu.SemaphoreType.DMA,
            pltpu.SemaphoreType.DMA,
            pltpu.SemaphoreType.DMA,
            pltpu.SemaphoreType.DMA,
        ],
        compiler_params=pltpu.CompilerParams(
            use_tc_tiling_on_sc=True, needs_layout_passes=False
        ),
    )
    def transpose_kernel(
        tt_hbm, tail_hbm, t4_hbm, inA, inB, outA, outB, isemA, isemB,
        osemA, osemB
    ):
        wid = lax.axis_index("s") * NC + lax.axis_index("c")
        iota = lax.iota(jnp.int32, 16)
        tvecs = [(iota + d) & 15 for d in range(16)]
        wbase = [t * 32 + iota for t in tvecs]

        def fire_in(gid, inb, isem):
            base = gid * 512
            for j in range(4):
                pltpu.async_copy(
                    tt_hbm.at[:, pl.ds(base + j * 128, 128)],
                    inb.at[pl.ds(j * 32, 32)],
                    isem,
                )

        def drain_in(gid, inb, isem):
            base = gid * 512
            for j in range(4):
                pltpu.make_async_copy(
                    tt_hbm.at[:, pl.ds(base + j * 128, 128)],
                    inb.at[pl.ds(j * 32, 32)],
                    isem,
                ).wait()

        def extract(inb, outb):
            def emit(m, carry):
                j = m >> 4
                h = (m >> 3) & 1
                e0 = (m & 7) * 16
                rows = j * 32 + h * 16 + iota
                wb = j * 4096 + e0 * 32 + h * 16
                for d0 in range(0, 16, 4):
                    vs = [
                        plsc.load_gather(inb, [rows, e0 + tvecs[d0 + u]])
                        for u in range(4)
                    ]
                    for u in range(4):
                        w = wb + wbase[d0 + u]
                        plsc.store_scatter(outb, [w >> 7, w & 127], vs[u])
                return carry

            lax.fori_loop(0, 64, emit, 0)

        def pair_body(i, carry):
            gidA = wid * per_w + 2 * i
            gidB = gidA + 1
            okA, okB = gidA < NG, gidB < NG

            @pl.when(okA)
            def _():
                fire_in(gidA, inA, isemA)

            @pl.when(okB)
            def _():
                fire_in(gidB, inB, isemB)

            @pl.when(okA)
            def _():
                drain_in(gidA, inA, isemA)
                extract(inA, outA)
                pltpu.async_copy(
                    outA, t4_hbm.at[pl.ds(gidA * 128, 128)], osemA
                )

            @pl.when(okB)
            def _():
                drain_in(gidB, inB, isemB)
                extract(inB, outB)
                pltpu.async_copy(
                    outB, t4_hbm.at[pl.ds(gidB * 128, 128)], osemB
                )

            @pl.when(okA)
            def _():
                pltpu.make_async_copy(
                    outA, t4_hbm.at[pl.ds(gidA * 128, 128)], osemA
                ).wait()

            @pl.when(okB)
            def _():
                pltpu.make_async_copy(
                    outB, t4_hbm.at[pl.ds(gidB * 128, 128)], osemB
                ).wait()

            return carry

        lax.fori_loop(0, per_w // 2, pair_body, 0)

        @pl.when(wid == NW - 1)
        def _():
            pltpu.sync_copy(tail_hbm, outA.at[pl.ds(0, n_tail)])
            pltpu.sync_copy(
                outA.at[pl.ds(0, n_tail)], t4_hbm.at[pl.ds(NG * 128, n_tail)]
            )

    return transpose_kernel(tableT, tail)


def _gather_rows(table, idx_flat):
    BF = idx_flat.shape[0]
    info = plsc.get_sparse_core_info()
    NC, NS = info.num_cores, info.num_subcores
    NW = NC * NS
    per_w = BF // NW
    C = 3328
    assert per_w % C == 0
    n_chunks = per_w // C
    mesh = plsc.VectorSubcoreMesh(core_axis_name="c", subcore_axis_name="s")

    @functools.partial(
        pl.kernel,
        out_type=jax.ShapeDtypeStruct((BF, _D), jnp.float32),
        mesh=mesh,
        scratch_types=[
            pltpu.VMEM((C,), jnp.int32),
            pltpu.VMEM((C, _D), jnp.float32),
            pltpu.SemaphoreType.DMA,
        ],
        compiler_params=pltpu.CompilerParams(use_tc_tiling_on_sc=False),
    )
    def gather_kernel(idx_hbm, table_hbm, out_hbm, idx_v, rows_v, sem):
        wid = lax.axis_index("s") * NC + lax.axis_index("c")
        for i in range(n_chunks):
            base = wid * per_w + i * C
            pltpu.sync_copy(idx_hbm.at[pl.ds(base, C)], idx_v)
            pltpu.async_copy(table_hbm.at[idx_v], rows_v, sem).wait()
            pltpu.sync_copy(rows_v, out_hbm.at[pl.ds(base, C)])

    return gather_kernel(idx_flat, table)


def _mlp_transposed(g4, W1d, b1d, W2d, b2t, F, B):
    Q = B // 4

    def body(g_ref, w1_ref, b1_ref, w2_ref, b2_ref, o_ref):
        g = g_ref[0]
        h = jnp.dot(g, w1_ref[...], preferred_element_type=jnp.float32)
        h = jnp.maximum(h + b1_ref[...], 0.0)
        ot = jax.lax.dot_general(
            w2_ref[...], h, (((0,), (1,)), ((), ())),
            preferred_element_type=jnp.float32,
        )
        ot = ot + b2_ref[...]
        for r in range(4):
            o_ref[0, :, r * Q : (r + 1) * Q] = ot[r * _H2 : (r + 1) * _H2, :]

    return pl.pallas_call(
        body,
        grid=(F,),
        in_specs=[
            pl.BlockSpec((1, Q, 4 * _D), lambda f: (f, 0, 0)),
            pl.BlockSpec((4 * _D, 4 * _H1), lambda f: (0, 0)),
            pl.BlockSpec((1, 4 * _H1), lambda f: (0, 0)),
            pl.BlockSpec((4 * _H1, 4 * _H2), lambda f: (0, 0)),
            pl.BlockSpec((4 * _H2, 1), lambda f: (0, 0)),
        ],
        out_specs=pl.BlockSpec((1, _H2, B), lambda f: (f, 0, 0)),
        out_shape=jax.ShapeDtypeStruct((F, _H2, B), jnp.float32),
    )(g4, W1d, b1d, W2d, b2t)


def _block_diag4(W):
    a, b = W.shape
    out = jnp.zeros((4 * a, 4 * b), dtype=W.dtype)
    for r in range(4):
        out = out.at[r * a : (r + 1) * a, r * b : (r + 1) * b].set(W)
    return out


def kernel(x, table, W1, b1, W2, b2):
    B, F = x.shape
    Q = B // 4
    idx_flat = (
        x.T.reshape(F, 4, Q).transpose(0, 2, 1).reshape(-1).astype(jnp.int32)
    )
    table_rm = _detile_table(table).reshape(table.shape[0], _D)
    g = _gather_rows(table_rm, idx_flat)
    g4 = g.reshape(F, Q, 4 * _D)
    W1d = _block_diag4(W1)
    W2d = _block_diag4(W2)
    b1d = jnp.tile(b1, 4).reshape(1, 4 * _H1)
    b2t = jnp.tile(b2, 4).reshape(4 * _H2, 1)
    o3 = _mlp_transposed(g4, W1d, b1d, W2d, b2t, F, B)
    return o3.transpose(2, 0, 1)


# speedup vs baseline: 1.7771x; 1.1565x over previous
import functools

import jax
import jax.numpy as jnp
from jax import lax
from jax.experimental import pallas as pl
from jax.experimental.pallas import tpu as pltpu
from jax.experimental.pallas import tpu_sc as plsc

_D = 32
_H1 = 64
_H2 = 32


def _detile_table(table):
    V = table.shape[0]
    VF = (V // 512) * 512
    tableT = table.T
    tail = table[VF:].reshape(((V - VF) * _D) // 128, 128)
    n_tail = tail.shape[0]
    NG = VF // 512
    R4 = (V * _D) // 128
    info = plsc.get_sparse_core_info()
    NC, NS = info.num_cores, info.num_subcores
    NW = NC * NS
    per_w = 2 * (-(-NG // (2 * NW)))
    mesh = plsc.VectorSubcoreMesh(core_axis_name="c", subcore_axis_name="s")

    @functools.partial(
        pl.kernel,
        out_type=jax.ShapeDtypeStruct((R4, 128), jnp.float32),
        mesh=mesh,
        scratch_types=[
            pltpu.VMEM((128, 128), jnp.float32),
            pltpu.VMEM((128, 128), jnp.float32),
            pltpu.VMEM((128, 128), jnp.float32),
            pltpu.VMEM((128, 128), jnp.float32),
            pltpu.SemaphoreType.DMA,
            pltpu.SemaphoreType.DMA,
            pltpu.SemaphoreType.DMA,
            pltpu.SemaphoreType.DMA,
        ],
        compiler_params=pltpu.CompilerParams(
            use_tc_tiling_on_sc=True, needs_layout_passes=False
        ),
    )
    def transpose_kernel(
        tt_hbm, tail_hbm, t4_hbm, inA, inB, outA, outB, isemA, isemB,
        osemA, osemB
    ):
        wid = lax.axis_index("s") * NC + lax.axis_index("c")
        iota = lax.iota(jnp.int32, 16)
        tvecs = [(iota + d) & 15 for d in range(16)]
        wbase = [t * 32 + iota for t in tvecs]

        def fire_in(gid, inb, isem):
            base = gid * 512
            for j in range(4):
                pltpu.async_copy(
                    tt_hbm.at[:, pl.ds(base + j * 128, 128)],
                    inb.at[pl.ds(j * 32, 32)],
                    isem,
                )

        def drain_in(gid, inb, isem):
            base = gid * 512
            for j in range(4):
                pltpu.make_async_copy(
                    tt_hbm.at[:, pl.ds(base + j * 128, 128)],
                    inb.at[pl.ds(j * 32, 32)],
                    isem,
                ).wait()

        def extract(inb, outb):
            def emit(m, carry):
                j = m >> 4
                h = (m >> 3) & 1
                e0 = (m & 7) * 16
                rows = j * 32 + h * 16 + iota
                wb = j * 4096 + e0 * 32 + h * 16
                for d0 in range(0, 16, 8):
                    vs = [
                        plsc.load_gather(inb, [rows, e0 + tvecs[d0 + u]])
                        for u in range(8)
                    ]
                    for u in range(4):
                        w = wb + wbase[d0 + u]
                        plsc.store_scatter(outb, [w >> 7, w & 127], vs[u])
                return carry

            lax.fori_loop(0, 64, emit, 0)

        def pair_body(i, carry):
            gidA = wid * per_w + 2 * i
            gidB = gidA + 1
            okA, okB = gidA < NG, gidB < NG

            @pl.when(okA)
            def _():
                fire_in(gidA, inA, isemA)

            @pl.when(okB)
            def _():
                fire_in(gidB, inB, isemB)

            @pl.when(okA)
            def _():
                drain_in(gidA, inA, isemA)
                extract(inA, outA)
                pltpu.async_copy(
                    outA, t4_hbm.at[pl.ds(gidA * 128, 128)], osemA
                )

            @pl.when(okB)
            def _():
                drain_in(gidB, inB, isemB)
                extract(inB, outB)
                pltpu.async_copy(
                    outB, t4_hbm.at[pl.ds(gidB * 128, 128)], osemB
                )

            @pl.when(okA)
            def _():
                pltpu.make_async_copy(
                    outA, t4_hbm.at[pl.ds(gidA * 128, 128)], osemA
                ).wait()

            @pl.when(okB)
            def _():
                pltpu.make_async_copy(
                    outB, t4_hbm.at[pl.ds(gidB * 128, 128)], osemB
                ).wait()

            return carry

        lax.fori_loop(0, per_w // 2, pair_body, 0)

        @pl.when(wid == NW - 1)
        def _():
            pltpu.sync_copy(tail_hbm, outA.at[pl.ds(0, n_tail)])
            pltpu.sync_copy(
                outA.at[pl.ds(0, n_tail)], t4_hbm.at[pl.ds(NG * 128, n_tail)]
            )

    return transpose_kernel(tableT, tail)


def _gather_rows(table, idx_flat):
    BF = idx_flat.shape[0]
    info = plsc.get_sparse_core_info()
    NC, NS = info.num_cores, info.num_subcores
    NW = NC * NS
    per_w = BF // NW
    C = 3328
    assert per_w % C == 0
    n_chunks = per_w // C
    mesh = plsc.VectorSubcoreMesh(core_axis_name="c", subcore_axis_name="s")

    @functools.partial(
        pl.kernel,
        out_type=jax.ShapeDtypeStruct((BF, _D), jnp.float32),
        mesh=mesh,
        scratch_types=[
            pltpu.VMEM((C,), jnp.int32),
            pltpu.VMEM((C, _D), jnp.float32),
            pltpu.SemaphoreType.DMA,
        ],
        compiler_params=pltpu.CompilerParams(use_tc_tiling_on_sc=False),
    )
    def gather_kernel(idx_hbm, table_hbm, out_hbm, idx_v, rows_v, sem):
        wid = lax.axis_index("s") * NC + lax.axis_index("c")
        for i in range(n_chunks):
            base = wid * per_w + i * C
            pltpu.sync_copy(idx_hbm.at[pl.ds(base, C)], idx_v)
            pltpu.async_copy(table_hbm.at[idx_v], rows_v, sem).wait()
            pltpu.sync_copy(rows_v, out_hbm.at[pl.ds(base, C)])

    return gather_kernel(idx_flat, table)


def _mlp_transposed(g4, W1d, b1d, W2d, b2t, F, B):
    Q = B // 4

    def body(g_ref, w1_ref, b1_ref, w2_ref, b2_ref, o_ref):
        g = g_ref[0]
        h = jnp.dot(g, w1_ref[...], preferred_element_type=jnp.float32)
        h = jnp.maximum(h + b1_ref[...], 0.0)
        ot = jax.lax.dot_general(
            w2_ref[...], h, (((0,), (1,)), ((), ())),
            preferred_element_type=jnp.float32,
        )
        ot = ot + b2_ref[...]
        for r in range(4):
            o_ref[0, :, r * Q : (r + 1) * Q] = ot[r * _H2 : (r + 1) * _H2, :]

    return pl.pallas_call(
        body,
        grid=(F,),
        in_specs=[
            pl.BlockSpec((1, Q, 4 * _D), lambda f: (f, 0, 0)),
            pl.BlockSpec((4 * _D, 4 * _H1), lambda f: (0, 0)),
            pl.BlockSpec((1, 4 * _H1), lambda f: (0, 0)),
            pl.BlockSpec((4 * _H1, 4 * _H2), lambda f: (0, 0)),
            pl.BlockSpec((4 * _H2, 1), lambda f: (0, 0)),
        ],
        out_specs=pl.BlockSpec((1, _H2, B), lambda f: (f, 0, 0)),
        out_shape=jax.ShapeDtypeStruct((F, _H2, B), jnp.float32),
    )(g4, W1d, b1d, W2d, b2t)


def _block_diag4(W):
    a, b = W.shape
    out = jnp.zeros((4 * a, 4 * b), dtype=W.dtype)
    for r in range(4):
        out = out.at[r * a : (r + 1) * a, r * b : (r + 1) * b].set(W)
    return out


def kernel(x, table, W1, b1, W2, b2):
    B, F = x.shape
    Q = B // 4
    idx_flat = (
        x.T.reshape(F, 4, Q).transpose(0, 2, 1).reshape(-1).astype(jnp.int32)
    )
    table_rm = _detile_table(table).reshape(table.shape[0], _D)
    g = _gather_rows(table_rm, idx_flat)
    g4 = g.reshape(F, Q, 4 * _D)
    W1d = _block_diag4(W1)
    W2d = _block_diag4(W2)
    b1d = jnp.tile(b1, 4).reshape(1, 4 * _H1)
    b2t = jnp.tile(b2, 4).reshape(4 * _H2, 1)
    o3 = _mlp_transposed(g4, W1d, b1d, W2d, b2t, F, B)
    return o3.transpose(2, 0, 1)
